# baseline (speedup 1.0000x reference)
import jax
import jax.numpy as jnp
from jax.experimental import pallas as pl
from jax.experimental.pallas import tpu as pltpu

_BI = 400
_CH = 40
_NC = _BI // _CH


def _gcn_kernel(features_ref, weight_ref, adj_hbm, out_ref,
                support_ref, buf_ref, sem_ref):
    i = pl.program_id(0)
    nsteps = pl.num_programs(0)

    def chunk_copy(block_idx, slot, c):
        return pltpu.make_async_copy(
            adj_hbm.at[pl.ds(block_idx * _BI + c * _CH, _CH), :],
            buf_ref.at[slot, pl.ds(c * _CH, _CH), :],
            sem_ref.at[slot, c],
        )

    @pl.when(i == 0)
    def _():
        for b in range(2):
            for c in range(_NC):
                chunk_copy(b, b, c).start()
        support_ref[...] = jnp.dot(
            features_ref[...].astype(jnp.bfloat16),
            weight_ref[...].astype(jnp.bfloat16),
            preferred_element_type=jnp.float32,
        ).astype(jnp.bfloat16)

    @pl.when(i + 2 < nsteps)
    def _():
        for c in range(_NC):
            chunk_copy(i + 2, (i + 2) % 3, c).start()

    slot = i % 3
    for c in range(_NC):
        chunk_copy(i, slot, c).wait()
    out_ref[...] = jnp.dot(
        buf_ref[slot].astype(jnp.bfloat16),
        support_ref[...],
        preferred_element_type=jnp.float32,
    )


def kernel(features, adj, weight):
    n, d_in = features.shape
    d_out = weight.shape[1]
    return pl.pallas_call(
        _gcn_kernel,
        grid=(n // _BI,),
        in_specs=[
            pl.BlockSpec((n, d_in), lambda i: (0, 0)),
            pl.BlockSpec((d_in, d_out), lambda i: (0, 0)),
            pl.BlockSpec(memory_space=pltpu.MemorySpace.HBM),
        ],
        out_specs=pl.BlockSpec((_BI, d_out), lambda i: (i, 0)),
        out_shape=jax.ShapeDtypeStruct((n, d_out), jnp.float32),
        scratch_shapes=[
            pltpu.VMEM((n, d_out), jnp.bfloat16),
            pltpu.VMEM((3, _BI, n), jnp.float32),
            pltpu.SemaphoreType.DMA((3, _NC)),
        ],
    )(features, weight, adj)


# speedup vs baseline: 1.0368x; 1.0368x over previous
import jax
import jax.numpy as jnp
from jax.experimental import pallas as pl
from jax.experimental.pallas import tpu as pltpu

_BI = 400


def _gcn_kernel(features_ref, weight_ref, adj_ref, out_ref, support_ref):
    i = pl.program_id(0)

    @pl.when(i == 0)
    def _():
        support_ref[...] = jnp.dot(
            features_ref[...].astype(jnp.bfloat16),
            weight_ref[...].astype(jnp.bfloat16),
            preferred_element_type=jnp.float32,
        ).astype(jnp.bfloat16)

    out_ref[...] = jnp.dot(
        adj_ref[...].astype(jnp.bfloat16),
        support_ref[...],
        preferred_element_type=jnp.float32,
    )


def kernel(features, adj, weight):
    n, d_in = features.shape
    d_out = weight.shape[1]
    return pl.pallas_call(
        _gcn_kernel,
        grid=((n + _BI - 1) // _BI,),
        in_specs=[
            pl.BlockSpec((n, d_in), lambda i: (0, 0)),
            pl.BlockSpec((d_in, d_out), lambda i: (0, 0)),
            pl.BlockSpec((_BI, n), lambda i: (i, 0)),
        ],
        out_specs=pl.BlockSpec((_BI, d_out), lambda i: (i, 0)),
        out_shape=jax.ShapeDtypeStruct((n, d_out), jnp.float32),
        scratch_shapes=[pltpu.VMEM((n, d_out), jnp.bfloat16)],
    )(features, weight, adj)
